# baseline (speedup 1.0000x reference)
import dataclasses
import functools

import jax
import jax.numpy as jnp
from jax import lax
from jax.experimental import pallas as pl
from jax.experimental.pallas import tpu as pltpu
from jax.experimental.pallas import tpu_sc as plsc

_NC = 2
_NS = 16
_CH = 80


def _sc_compiler_params():
    cp = pltpu.CompilerParams()
    if "needs_layout_passes" in pltpu.CompilerParams.__dataclass_fields__:
        cp = dataclasses.replace(cp, needs_layout_passes=False)
    return cp


def _sc_edge_kernel(n, e, d):
    nw = _NC * _NS
    ep = e // nw
    nch = ep // _CH
    assert ep % _CH == 0 and nch % 3 == 2 and nch >= 5
    stride = ((n // _NS) // 8) * 8
    stripe = n - stride * (_NS - 1)
    assert stripe >= stride and stripe % 8 == 0

    mesh = plsc.VectorSubcoreMesh(core_axis_name="c", subcore_axis_name="s")
    f32 = jnp.float32
    i32 = jnp.int32

    scratch = (
        [pltpu.VMEM((_CH,), i32)] * 24
        + [pltpu.VMEM((_CH,), f32)] * 9
        + [pltpu.VMEM((_CH, d), f32)] * 3
        + [pltpu.VMEM((16,), i32)]
        + [pltpu.VMEM_SHARED((n, d), f32), pltpu.VMEM_SHARED((n,), f32)]
        + [pltpu.SemaphoreType.DMA] * 12
    )

    @functools.partial(
        pl.kernel,
        out_type=[
            jax.ShapeDtypeStruct((e,), f32),
            jax.ShapeDtypeStruct((e,), f32),
            jax.ShapeDtypeStruct((_NC, n, d), f32),
            jax.ShapeDtypeStruct((_NC, n), f32),
        ],
        mesh=mesh,
        scratch_types=scratch,
        compiler_params=_sc_compiler_params(),
    )
    def sc_kernel(maskf, x, hrow, hcol, hval, vrow, vcol, vval, nid, zu, zs,
                  horw, verw, upart, spart, *scr):
        rowc = scr[0:3]
        colc = scr[3:6]
        i1 = scr[6:9]
        i2 = scr[9:12]
        rowIdx = scr[12:15]
        colIdx = scr[15:18]
        m1b = scr[18:21]
        m2b = scr[21:24]
        valc = scr[24:27]
        wb = scr[27:30]
        selw = scr[30:33]
        rowsb = scr[33:36]
        nid_b = scr[36]
        u_acc, s_acc = scr[37], scr[38]
        semL = scr[39:42]
        semG = scr[42:45]
        semS = scr[45:48]
        semA = scr[48:51]

        c = lax.axis_index("c")
        sid = lax.axis_index("s")
        wid = c * _NS + sid
        base = pl.multiple_of(wid * ep, 8)

        r0 = pl.multiple_of(sid * stride, 8)
        pltpu.async_copy(nid, nid_b, semL[0])
        pltpu.sync_copy(zu.at[pl.ds(r0, stripe)], u_acc.at[pl.ds(r0, stripe)])

        @pl.when(sid == 0)
        def _():
            pltpu.sync_copy(zs, s_acc)

        pltpu.make_async_copy(nid, nid_b, semL[0]).wait()
        plsc.subcore_barrier()

        def chunk_slice(k):
            return pl.ds(pl.multiple_of(base + k * _CH, 8), _CH)

        def run_phase(rowr, colr, valr, wout, with_rows):
            def load(k, p):
                sl = chunk_slice(k)
                pltpu.async_copy(rowr.at[sl], rowc[p], semL[p])
                pltpu.async_copy(colr.at[sl], colc[p], semL[p])
                pltpu.async_copy(valr.at[sl], valc[p], semL[p])

            def prep(k, p):
                sl = chunk_slice(k)
                pltpu.make_async_copy(rowr.at[sl], rowc[p], semL[p]).wait()
                pltpu.make_async_copy(colr.at[sl], colc[p], semL[p]).wait()
                pltpu.make_async_copy(valr.at[sl], valc[p], semL[p]).wait()

                @pl.loop(0, _CH, step=16)
                def _(j):
                    js = pl.ds(j, 16)
                    r = rowc[p][js]
                    cc = colc[p][js]
                    i1[p][js] = r * (n // 2) + lax.shift_right_logical(cc, 1)
                    i2[p][js] = cc * (n // 2) + lax.shift_right_logical(r, 1)
                    colIdx[p][js] = cc
                    if with_rows:
                        rowIdx[p][js] = r

                pltpu.async_copy(maskf.at[i1[p]], m1b[p], semG[p])
                pltpu.async_copy(maskf.at[i2[p]], m2b[p], semG[p])
                if with_rows:
                    pltpu.async_copy(x.at[colIdx[p]], rowsb[p], semG[p])

            def process(k, p, store_guard):
                pltpu.make_async_copy(maskf.at[i1[p]], m1b[p], semG[p]).wait()
                pltpu.make_async_copy(maskf.at[i2[p]], m2b[p], semG[p]).wait()
                if with_rows:
                    pltpu.make_async_copy(x.at[colIdx[p]], rowsb[p],
                                          semG[p]).wait()

                def drain_store():
                    pltpu.make_async_copy(wb[p], wout.at[pl.ds(0, _CH)],
                                          semS[p]).wait()

                if store_guard is True:
                    drain_store()
                else:
                    pl.when(store_guard)(drain_store)

                @pl.loop(0, _CH, step=16)
                def _(j):
                    js = pl.ds(j, 16)

                    def unpack(wordv, parity):
                        bits = jnp.where(parity == 1,
                                         lax.shift_right_logical(wordv, 16),
                                         wordv)
                        return plsc.bitcast(
                            lax.shift_left(bits & 0xFFFF, 16), f32)

                    f1 = unpack(m1b[p][js], colc[p][js] & 1)
                    f2 = unpack(m2b[p][js], rowc[p][js] & 1)
                    sg1 = 1.0 / (1.0 + jnp.exp(-f1))
                    sg2 = 1.0 / (1.0 + jnp.exp(-f2))
                    wb[p][js] = valc[p][js] * ((sg1 + sg2) * 0.5)

                pltpu.async_copy(wb[p], wout.at[chunk_slice(k)], semS[p])

                if with_rows:
                    @pl.loop(0, _CH, step=16)
                    def _(j):
                        wv = wb[p][pl.ds(j, 16)]
                        for i in range(16):
                            ws = wv[i]
                            for jj in range(d // 16):
                                rs = pl.ds(jj * 16, 16)
                                rowsb[p][j + i, rs] = rowsb[p][j + i, rs] * ws

                    pltpu.async_copy(rowsb[p], u_acc.at[rowIdx[p]], semA[p],
                                     add=True)
                else:
                    nv = nid_b[...]

                    @pl.loop(0, _CH, step=16)
                    def _(j):
                        js = pl.ds(j, 16)
                        sel = rowc[p][js] == nv
                        selw[p][js] = jnp.where(sel, wb[p][js], 0.0)

                    pltpu.async_copy(selw[p], s_acc.at[colIdx[p]], semA[p],
                                     add=True)

            def drain(p):
                if with_rows:
                    pltpu.make_async_copy(rowsb[p], u_acc.at[rowIdx[p]],
                                          semA[p]).wait()
                else:
                    pltpu.make_async_copy(selw[p], s_acc.at[colIdx[p]],
                                          semA[p]).wait()

            load(0, 0)
            load(1, 1)
            prep(0, 0)

            @pl.loop(0, nch - 2, step=3)
            def _(k):
                for q in range(3):
                    kk = k + q
                    load(kk + 2, (q + 2) % 3)
                    prep(kk + 1, (q + 1) % 3)
                    process(kk, q, kk >= 3)
                    pr = (q + 2) % 3

                    @pl.when(kk > 0)
                    def _():
                        drain(pr)

            prep(nch - 1, (nch - 1) % 3)
            process(nch - 2, (nch - 2) % 3, True)
            drain((nch - 3) % 3)
            process(nch - 1, (nch - 1) % 3, True)
            drain((nch - 2) % 3)
            drain((nch - 1) % 3)
            for p in range(3):
                pltpu.make_async_copy(wb[p], wout.at[pl.ds(0, _CH)],
                                      semS[p]).wait()

        run_phase(hrow, hcol, hval, horw, True)
        run_phase(vrow, vcol, vval, verw, False)

        plsc.subcore_barrier()
        pltpu.sync_copy(u_acc.at[pl.ds(r0, stripe)],
                        upart.at[c].at[pl.ds(r0, stripe)])

        @pl.when(sid == 0)
        def _():
            pltpu.sync_copy(s_acc, spart.at[c])

    return sc_kernel


def _tc_tail(u_part, s_part, W1, W2):
    n = u_part.shape[1]
    cdim = W2.shape[1]

    def body(u_ref, s_ref, w1_ref, w2_ref, o_ref):
        u = u_ref[0] + u_ref[1]
        g = jnp.dot(u, w1_ref[...], preferred_element_type=jnp.float32)
        g = jnp.maximum(g, 0.0)
        s = (s_ref[0] + s_ref[1]).reshape(n, 1)
        t = jnp.sum(g * s, axis=0, keepdims=True)
        p = jnp.dot(t, w2_ref[...], preferred_element_type=jnp.float32)
        m = jnp.max(p)
        ex = jnp.exp(p - m)
        o_ref[...] = ex / jnp.sum(ex)

    out = pl.pallas_call(
        body,
        out_shape=jax.ShapeDtypeStruct((1, cdim), jnp.float32),
    )(u_part, s_part, W1, W2)
    return out.reshape(cdim)


def kernel(mask, x, W1, W2, hor_values, hor_indices, ver_values, ver_indices,
           node_idx):
    n = mask.shape[0]
    e = hor_values.shape[0]
    d = x.shape[1]

    assert n % 2 == 0
    mu = jax.lax.bitcast_convert_type(mask.astype(jnp.bfloat16), jnp.uint16)
    lo = mu[:, 0::2].astype(jnp.int32)
    hi = mu[:, 1::2].astype(jnp.int32)
    maskf = (lo | jax.lax.shift_left(hi, 16)).reshape(n * (n // 2))
    nid = jnp.full((16,), node_idx, dtype=jnp.int32)
    zu = jnp.zeros((n, d), jnp.float32)
    zs = jnp.zeros((n,), jnp.float32)

    sc = _sc_edge_kernel(n, e, d)
    hor_w, ver_w, u_part, s_part = sc(
        maskf, x,
        hor_indices[0], hor_indices[1], hor_values,
        ver_indices[0], ver_indices[1], ver_values,
        nid, zu, zs)

    res = _tc_tail(u_part, s_part, W1, W2)
    return (res, hor_w, ver_w)


# speedup vs baseline: 23.2607x; 23.2607x over previous
import dataclasses
import functools

import jax
import jax.numpy as jnp
from jax import lax
from jax.experimental import pallas as pl
from jax.experimental.pallas import tpu as pltpu
from jax.experimental.pallas import tpu_sc as plsc

_NC = 2
_NS = 16
_CH = 80


def _sc_compiler_params():
    cp = pltpu.CompilerParams()
    if "needs_layout_passes" in pltpu.CompilerParams.__dataclass_fields__:
        cp = dataclasses.replace(cp, needs_layout_passes=False)
    return cp


def _sc_edge_kernel(n, e, d):
    nw = _NC * _NS
    ep = e // nw
    nch = ep // _CH
    assert ep % _CH == 0 and nch % 3 == 2 and nch >= 5
    stride = ((n // _NS) // 8) * 8
    stripe = n - stride * (_NS - 1)
    assert stripe >= stride and stripe % 8 == 0

    mesh = plsc.VectorSubcoreMesh(core_axis_name="c", subcore_axis_name="s")
    f32 = jnp.float32
    i32 = jnp.int32

    scratch = (
        [pltpu.VMEM((_CH,), i32)] * 18
        + [pltpu.VMEM((_CH,), f32)] * 15
        + [pltpu.VMEM((_CH, d), f32)] * 3
        + [pltpu.VMEM((16,), i32)]
        + [pltpu.VMEM_SHARED((n, d), f32), pltpu.VMEM_SHARED((n,), f32)]
        + [pltpu.SemaphoreType.DMA] * 12
    )

    @functools.partial(
        pl.kernel,
        out_type=[
            jax.ShapeDtypeStruct((e,), f32),
            jax.ShapeDtypeStruct((e,), f32),
            jax.ShapeDtypeStruct((_NC, n, d), f32),
            jax.ShapeDtypeStruct((_NC, n), f32),
        ],
        mesh=mesh,
        scratch_types=scratch,
        compiler_params=_sc_compiler_params(),
    )
    def sc_kernel(maskf, x, hrow, hcol, hval, vrow, vcol, vval, nid, zu, zs,
                  horw, verw, upart, spart, *scr):
        rowc = scr[0:3]
        colc = scr[3:6]
        i1 = scr[6:9]
        i2 = scr[9:12]
        rowIdx = scr[12:15]
        colIdx = scr[15:18]
        m1b = scr[18:21]
        m2b = scr[21:24]
        valc = scr[24:27]
        wb = scr[27:30]
        selw = scr[30:33]
        rowsb = scr[33:36]
        nid_b = scr[36]
        u_acc, s_acc = scr[37], scr[38]
        semL = scr[39:42]
        semG = scr[42:45]
        semS = scr[45:48]
        semA = scr[48:51]

        c = lax.axis_index("c")
        sid = lax.axis_index("s")
        wid = c * _NS + sid
        base = pl.multiple_of(wid * ep, 8)

        r0 = pl.multiple_of(sid * stride, 8)
        pltpu.async_copy(nid, nid_b, semL[0])
        pltpu.sync_copy(zu.at[pl.ds(r0, stripe)], u_acc.at[pl.ds(r0, stripe)])

        @pl.when(sid == 0)
        def _():
            pltpu.sync_copy(zs, s_acc)

        pltpu.make_async_copy(nid, nid_b, semL[0]).wait()
        plsc.subcore_barrier()

        def chunk_slice(k):
            return pl.ds(pl.multiple_of(base + k * _CH, 8), _CH)

        def run_phase(rowr, colr, valr, wout, with_rows):
            def load(k, p):
                sl = chunk_slice(k)
                pltpu.async_copy(rowr.at[sl], rowc[p], semL[p])
                pltpu.async_copy(colr.at[sl], colc[p], semL[p])
                pltpu.async_copy(valr.at[sl], valc[p], semL[p])

            def prep(k, p):
                sl = chunk_slice(k)
                pltpu.make_async_copy(rowr.at[sl], rowc[p], semL[p]).wait()
                pltpu.make_async_copy(colr.at[sl], colc[p], semL[p]).wait()
                pltpu.make_async_copy(valr.at[sl], valc[p], semL[p]).wait()

                @pl.loop(0, _CH, step=16)
                def _(j):
                    js = pl.ds(j, 16)
                    r = rowc[p][js]
                    cc = colc[p][js]
                    i1[p][js] = r * n + cc
                    i2[p][js] = cc * n + r
                    colIdx[p][js] = cc
                    if with_rows:
                        rowIdx[p][js] = r

                pltpu.async_copy(maskf.at[i1[p]], m1b[p], semG[p])
                pltpu.async_copy(maskf.at[i2[p]], m2b[p], semG[p])
                if with_rows:
                    pltpu.async_copy(x.at[colIdx[p]], rowsb[p], semG[p])

            def process(k, p, store_guard):
                pltpu.make_async_copy(maskf.at[i1[p]], m1b[p], semG[p]).wait()
                pltpu.make_async_copy(maskf.at[i2[p]], m2b[p], semG[p]).wait()
                if with_rows:
                    pltpu.make_async_copy(x.at[colIdx[p]], rowsb[p],
                                          semG[p]).wait()

                def drain_store():
                    pltpu.make_async_copy(wb[p], wout.at[pl.ds(0, _CH)],
                                          semS[p]).wait()

                if store_guard is True:
                    drain_store()
                else:
                    pl.when(store_guard)(drain_store)

                @pl.loop(0, _CH, step=16)
                def _(j):
                    js = pl.ds(j, 16)
                    sg1 = 1.0 / (1.0 + jnp.exp(-m1b[p][js]))
                    sg2 = 1.0 / (1.0 + jnp.exp(-m2b[p][js]))
                    wb[p][js] = valc[p][js] * ((sg1 + sg2) * 0.5)

                pltpu.async_copy(wb[p], wout.at[chunk_slice(k)], semS[p])

                if with_rows:
                    @pl.loop(0, _CH, step=16)
                    def _(j):
                        wv = wb[p][pl.ds(j, 16)]
                        for i in range(16):
                            ws = wv[i]
                            for jj in range(d // 16):
                                rs = pl.ds(jj * 16, 16)
                                rowsb[p][j + i, rs] = rowsb[p][j + i, rs] * ws

                    pltpu.async_copy(rowsb[p], u_acc.at[rowIdx[p]], semA[p],
                                     add=True)
                else:
                    nv = nid_b[...]

                    @pl.loop(0, _CH, step=16)
                    def _(j):
                        js = pl.ds(j, 16)
                        sel = rowc[p][js] == nv
                        selw[p][js] = jnp.where(sel, wb[p][js], 0.0)

                    pltpu.async_copy(selw[p], s_acc.at[colIdx[p]], semA[p],
                                     add=True)

            def drain(p):
                if with_rows:
                    pltpu.make_async_copy(rowsb[p], u_acc.at[rowIdx[p]],
                                          semA[p]).wait()
                else:
                    pltpu.make_async_copy(selw[p], s_acc.at[colIdx[p]],
                                          semA[p]).wait()

            load(0, 0)
            load(1, 1)
            prep(0, 0)

            @pl.loop(0, nch - 2, step=3)
            def _(k):
                for q in range(3):
                    kk = k + q
                    load(kk + 2, (q + 2) % 3)
                    prep(kk + 1, (q + 1) % 3)
                    process(kk, q, kk >= 3)
                    pr = (q + 2) % 3

                    @pl.when(kk > 0)
                    def _():
                        drain(pr)

            prep(nch - 1, (nch - 1) % 3)
            process(nch - 2, (nch - 2) % 3, True)
            drain((nch - 3) % 3)
            process(nch - 1, (nch - 1) % 3, True)
            drain((nch - 2) % 3)
            drain((nch - 1) % 3)
            for p in range(3):
                pltpu.make_async_copy(wb[p], wout.at[pl.ds(0, _CH)],
                                      semS[p]).wait()

        run_phase(hrow, hcol, hval, horw, True)
        run_phase(vrow, vcol, vval, verw, False)

        plsc.subcore_barrier()
        pltpu.sync_copy(u_acc.at[pl.ds(r0, stripe)],
                        upart.at[c].at[pl.ds(r0, stripe)])

        @pl.when(sid == 0)
        def _():
            pltpu.sync_copy(s_acc, spart.at[c])

    return sc_kernel


def _tc_tail(u_part, s_part, W1, W2):
    n = u_part.shape[1]
    cdim = W2.shape[1]

    def body(u_ref, s_ref, w1_ref, w2_ref, o_ref):
        u = u_ref[0] + u_ref[1]
        g = jnp.dot(u, w1_ref[...], preferred_element_type=jnp.float32)
        g = jnp.maximum(g, 0.0)
        s = (s_ref[0] + s_ref[1]).reshape(n, 1)
        t = jnp.sum(g * s, axis=0, keepdims=True)
        p = jnp.dot(t, w2_ref[...], preferred_element_type=jnp.float32)
        m = jnp.max(p)
        ex = jnp.exp(p - m)
        o_ref[...] = ex / jnp.sum(ex)

    out = pl.pallas_call(
        body,
        out_shape=jax.ShapeDtypeStruct((1, cdim), jnp.float32),
    )(u_part, s_part, W1, W2)
    return out.reshape(cdim)


def kernel(mask, x, W1, W2, hor_values, hor_indices, ver_values, ver_indices,
           node_idx):
    n = mask.shape[0]
    e = hor_values.shape[0]
    d = x.shape[1]

    maskf = mask.reshape(n * n)
    nid = jnp.full((16,), node_idx, dtype=jnp.int32)
    zu = jnp.zeros((n, d), jnp.float32)
    zs = jnp.zeros((n,), jnp.float32)

    sc = _sc_edge_kernel(n, e, d)
    hor_w, ver_w, u_part, s_part = sc(
        maskf, x,
        hor_indices[0], hor_indices[1], hor_values,
        ver_indices[0], ver_indices[1], ver_values,
        nid, zu, zs)

    res = _tc_tail(u_part, s_part, W1, W2)
    return (res, hor_w, ver_w)


# speedup vs baseline: 23.7005x; 1.0189x over previous
import dataclasses
import functools

import jax
import jax.numpy as jnp
from jax import lax
from jax.experimental import pallas as pl
from jax.experimental.pallas import tpu as pltpu
from jax.experimental.pallas import tpu_sc as plsc

_NC = 2
_NS = 16
_CH = 80


def _sc_compiler_params():
    cp = pltpu.CompilerParams()
    if "needs_layout_passes" in pltpu.CompilerParams.__dataclass_fields__:
        cp = dataclasses.replace(cp, needs_layout_passes=False)
    return cp


def _sc_edge_kernel(n, e, d):
    nw = _NC * _NS
    ep = e // nw
    nch = ep // _CH
    assert ep % _CH == 0 and nch % 3 == 2 and nch >= 5
    stride = ((n // _NS) // 8) * 8
    stripe = n - stride * (_NS - 1)
    assert stripe >= stride and stripe % 8 == 0

    mesh = plsc.VectorSubcoreMesh(core_axis_name="c", subcore_axis_name="s")
    f32 = jnp.float32
    i32 = jnp.int32

    scratch = (
        [pltpu.VMEM((_CH,), i32)] * 18
        + [pltpu.VMEM((_CH,), f32)] * 15
        + [pltpu.VMEM((_CH, d), f32)] * 3
        + [pltpu.VMEM((16,), i32)]
        + [pltpu.VMEM_SHARED((n, d), f32), pltpu.VMEM_SHARED((n,), f32)]
        + [pltpu.SemaphoreType.DMA] * 12
    )

    @functools.partial(
        pl.kernel,
        out_type=[
            jax.ShapeDtypeStruct((e,), f32),
            jax.ShapeDtypeStruct((e,), f32),
            jax.ShapeDtypeStruct((_NC, n, d), f32),
            jax.ShapeDtypeStruct((_NC, n), f32),
        ],
        mesh=mesh,
        scratch_types=scratch,
        compiler_params=_sc_compiler_params(),
    )
    def sc_kernel(maskf, x, hrow, hcol, hval, vrow, vcol, vval, nid,
                  horw, verw, upart, spart, *scr):
        rowc = scr[0:3]
        colc = scr[3:6]
        i1 = scr[6:9]
        i2 = scr[9:12]
        rowIdx = scr[12:15]
        colIdx = scr[15:18]
        m1b = scr[18:21]
        m2b = scr[21:24]
        valc = scr[24:27]
        wb = scr[27:30]
        selw = scr[30:33]
        rowsb = scr[33:36]
        nid_b = scr[36]
        u_acc, s_acc = scr[37], scr[38]
        semL = scr[39:42]
        semG = scr[42:45]
        semS = scr[45:48]
        semA = scr[48:51]

        c = lax.axis_index("c")
        sid = lax.axis_index("s")
        wid = c * _NS + sid
        base = pl.multiple_of(wid * ep, 8)

        r0 = pl.multiple_of(sid * stride, 8)
        pltpu.async_copy(nid, nid_b, semL[0])

        @pl.loop(0, _CH)
        def _(r):
            for jj in range(d // 16):
                rowsb[0][r, pl.ds(jj * 16, 16)] = jnp.zeros((16,), f32)

        @pl.loop(0, _CH, step=16)
        def _(j):
            selw[0][pl.ds(j, 16)] = jnp.zeros((16,), f32)

        assert stripe % _CH == 0
        for q in range(stripe // _CH):
            off = pl.multiple_of(r0 + q * _CH, 8)
            pltpu.sync_copy(rowsb[0], u_acc.at[pl.ds(off, _CH)])
            pltpu.sync_copy(selw[0], s_acc.at[pl.ds(off, _CH)])

        pltpu.make_async_copy(nid, nid_b, semL[0]).wait()
        plsc.subcore_barrier()

        def chunk_slice(k):
            return pl.ds(pl.multiple_of(base + k * _CH, 8), _CH)

        def run_phase(rowr, colr, valr, wout, with_rows):
            def load(k, p):
                sl = chunk_slice(k)
                pltpu.async_copy(rowr.at[sl], rowc[p], semL[p])
                pltpu.async_copy(colr.at[sl], colc[p], semL[p])
                pltpu.async_copy(valr.at[sl], valc[p], semL[p])

            def prep(k, p):
                sl = chunk_slice(k)
                pltpu.make_async_copy(rowr.at[sl], rowc[p], semL[p]).wait()
                pltpu.make_async_copy(colr.at[sl], colc[p], semL[p]).wait()
                pltpu.make_async_copy(valr.at[sl], valc[p], semL[p]).wait()

                @pl.loop(0, _CH, step=16)
                def _(j):
                    js = pl.ds(j, 16)
                    r = rowc[p][js]
                    cc = colc[p][js]
                    i1[p][js] = r * n + cc
                    i2[p][js] = cc * n + r
                    colIdx[p][js] = cc
                    if with_rows:
                        rowIdx[p][js] = r

                pltpu.async_copy(maskf.at[i1[p]], m1b[p], semG[p])
                pltpu.async_copy(maskf.at[i2[p]], m2b[p], semG[p])
                if with_rows:
                    pltpu.async_copy(x.at[colIdx[p]], rowsb[p], semG[p])

            def process(k, p, store_guard):
                pltpu.make_async_copy(maskf.at[i1[p]], m1b[p], semG[p]).wait()
                pltpu.make_async_copy(maskf.at[i2[p]], m2b[p], semG[p]).wait()

                def drain_store():
                    pltpu.make_async_copy(wb[p], wout.at[pl.ds(0, _CH)],
                                          semS[p]).wait()

                if store_guard is True:
                    drain_store()
                else:
                    pl.when(store_guard)(drain_store)

                @pl.loop(0, _CH, step=16)
                def _(j):
                    js = pl.ds(j, 16)
                    sg1 = 1.0 / (1.0 + jnp.exp(-m1b[p][js]))
                    sg2 = 1.0 / (1.0 + jnp.exp(-m2b[p][js]))
                    wb[p][js] = valc[p][js] * ((sg1 + sg2) * 0.5)

                pltpu.async_copy(wb[p], wout.at[chunk_slice(k)], semS[p])

                if with_rows:
                    pltpu.make_async_copy(x.at[colIdx[p]], rowsb[p],
                                          semG[p]).wait()

                    @pl.loop(0, _CH, step=16)
                    def _(j):
                        wv = wb[p][pl.ds(j, 16)]
                        for i in range(16):
                            ws = wv[i]
                            for jj in range(d // 16):
                                rs = pl.ds(jj * 16, 16)
                                rowsb[p][j + i, rs] = rowsb[p][j + i, rs] * ws

                    pltpu.async_copy(rowsb[p], u_acc.at[rowIdx[p]], semA[p],
                                     add=True)
                else:
                    nv = nid_b[...]

                    @pl.loop(0, _CH, step=16)
                    def _(j):
                        js = pl.ds(j, 16)
                        sel = rowc[p][js] == nv
                        selw[p][js] = jnp.where(sel, wb[p][js], 0.0)

                    pltpu.async_copy(selw[p], s_acc.at[colIdx[p]], semA[p],
                                     add=True)

            def drain(p):
                if with_rows:
                    pltpu.make_async_copy(rowsb[p], u_acc.at[rowIdx[p]],
                                          semA[p]).wait()
                else:
                    pltpu.make_async_copy(selw[p], s_acc.at[colIdx[p]],
                                          semA[p]).wait()

            load(0, 0)
            load(1, 1)
            prep(0, 0)

            @pl.loop(0, nch - 2, step=3)
            def _(k):
                for q in range(3):
                    kk = k + q
                    load(kk + 2, (q + 2) % 3)
                    prep(kk + 1, (q + 1) % 3)
                    process(kk, q, kk >= 3)
                    pr = (q + 2) % 3

                    @pl.when(kk > 0)
                    def _():
                        drain(pr)

            prep(nch - 1, (nch - 1) % 3)
            process(nch - 2, (nch - 2) % 3, True)
            drain((nch - 3) % 3)
            process(nch - 1, (nch - 1) % 3, True)
            drain((nch - 2) % 3)
            drain((nch - 1) % 3)
            for p in range(3):
                pltpu.make_async_copy(wb[p], wout.at[pl.ds(0, _CH)],
                                      semS[p]).wait()

        run_phase(hrow, hcol, hval, horw, True)
        run_phase(vrow, vcol, vval, verw, False)

        plsc.subcore_barrier()
        pltpu.sync_copy(u_acc.at[pl.ds(r0, stripe)],
                        upart.at[c].at[pl.ds(r0, stripe)])

        @pl.when(sid == 0)
        def _():
            pltpu.sync_copy(s_acc, spart.at[c])

    return sc_kernel


def _tc_tail(u_part, s_part, W1, W2):
    n = u_part.shape[1]
    cdim = W2.shape[1]

    def body(u_ref, s_ref, w1_ref, w2_ref, o_ref):
        u = u_ref[0] + u_ref[1]
        g = jnp.dot(u, w1_ref[...], preferred_element_type=jnp.float32)
        g = jnp.maximum(g, 0.0)
        s = (s_ref[0] + s_ref[1]).reshape(n, 1)
        t = jnp.sum(g * s, axis=0, keepdims=True)
        p = jnp.dot(t, w2_ref[...], preferred_element_type=jnp.float32)
        m = jnp.max(p)
        ex = jnp.exp(p - m)
        o_ref[...] = ex / jnp.sum(ex)

    out = pl.pallas_call(
        body,
        out_shape=jax.ShapeDtypeStruct((1, cdim), jnp.float32),
    )(u_part, s_part, W1, W2)
    return out.reshape(cdim)


def kernel(mask, x, W1, W2, hor_values, hor_indices, ver_values, ver_indices,
           node_idx):
    n = mask.shape[0]
    e = hor_values.shape[0]
    d = x.shape[1]

    maskf = mask.reshape(n * n)
    nid = jnp.full((16,), node_idx, dtype=jnp.int32)

    sc = _sc_edge_kernel(n, e, d)
    hor_w, ver_w, u_part, s_part = sc(
        maskf, x,
        hor_indices[0], hor_indices[1], hor_values,
        ver_indices[0], ver_indices[1], ver_values,
        nid)

    res = _tc_tail(u_part, s_part, W1, W2)
    return (res, hor_w, ver_w)


# speedup vs baseline: 24.1684x; 1.0197x over previous
import dataclasses
import functools

import jax
import jax.numpy as jnp
from jax import lax
from jax.experimental import pallas as pl
from jax.experimental.pallas import tpu as pltpu
from jax.experimental.pallas import tpu_sc as plsc

_NC = 2
_NS = 16
_CH = 80


def _sc_compiler_params():
    cp = pltpu.CompilerParams()
    if "needs_layout_passes" in pltpu.CompilerParams.__dataclass_fields__:
        cp = dataclasses.replace(cp, needs_layout_passes=False)
    return cp


def _sc_edge_kernel(n, e, d):
    nw = _NC * _NS
    ep = e // nw
    nch = ep // _CH
    assert ep % _CH == 0 and nch % 3 == 2 and nch >= 5
    stride = ((n // _NS) // 8) * 8
    stripe = n - stride * (_NS - 1)
    assert stripe >= stride and stripe % 8 == 0

    mesh = plsc.VectorSubcoreMesh(core_axis_name="c", subcore_axis_name="s")
    f32 = jnp.float32
    i32 = jnp.int32

    scratch = (
        [pltpu.VMEM((_CH,), i32)] * 18
        + [pltpu.VMEM((_CH,), f32)] * 15
        + [pltpu.VMEM((_CH, d), f32)] * 3
        + [pltpu.VMEM((16,), i32)]
        + [pltpu.VMEM_SHARED((n, d), f32), pltpu.VMEM_SHARED((n,), f32)]
        + [pltpu.SemaphoreType.DMA] * 12
    )

    @functools.partial(
        pl.kernel,
        out_type=[
            jax.ShapeDtypeStruct((e,), f32),
            jax.ShapeDtypeStruct((e,), f32),
            jax.ShapeDtypeStruct((_NC, n, d), f32),
            jax.ShapeDtypeStruct((_NC, n), f32),
        ],
        mesh=mesh,
        scratch_types=scratch,
        compiler_params=_sc_compiler_params(),
    )
    def sc_kernel(maskf, x, hidx, hval, vidx, vval, nid,
                  horw, verw, upart, spart, *scr):
        rowc = scr[0:3]
        colc = scr[3:6]
        i1 = scr[6:9]
        i2 = scr[9:12]
        rowIdx = scr[12:15]
        colIdx = scr[15:18]
        m1b = scr[18:21]
        m2b = scr[21:24]
        valc = scr[24:27]
        wb = scr[27:30]
        selw = scr[30:33]
        rowsb = scr[33:36]
        nid_b = scr[36]
        u_acc, s_acc = scr[37], scr[38]
        semL = scr[39:42]
        semG = scr[42:45]
        semS = scr[45:48]
        semA = scr[48:51]

        c = lax.axis_index("c")
        sid = lax.axis_index("s")
        wid = c * _NS + sid
        base = pl.multiple_of(wid * ep, 8)

        r0 = pl.multiple_of(sid * stride, 8)
        pltpu.async_copy(nid, nid_b, semL[0])

        @pl.loop(0, _CH)
        def _(r):
            for jj in range(d // 16):
                rowsb[0][r, pl.ds(jj * 16, 16)] = jnp.zeros((16,), f32)

        @pl.loop(0, _CH, step=16)
        def _(j):
            selw[0][pl.ds(j, 16)] = jnp.zeros((16,), f32)

        assert stripe % _CH == 0
        for q in range(stripe // _CH):
            off = pl.multiple_of(r0 + q * _CH, 8)
            pltpu.sync_copy(rowsb[0], u_acc.at[pl.ds(off, _CH)])
            pltpu.sync_copy(selw[0], s_acc.at[pl.ds(off, _CH)])

        pltpu.make_async_copy(nid, nid_b, semL[0]).wait()
        plsc.subcore_barrier()

        def chunk_slice(k):
            return pl.ds(pl.multiple_of(base + k * _CH, 8), _CH)

        def run_phase(idxr, valr, wout, with_rows):
            def col_slice(k):
                return pl.ds(pl.multiple_of(e + base + k * _CH, 8), _CH)

            def load(k, p):
                sl = chunk_slice(k)
                pltpu.async_copy(idxr.at[sl], rowc[p], semL[p])
                pltpu.async_copy(idxr.at[col_slice(k)], colc[p], semL[p])
                pltpu.async_copy(valr.at[sl], valc[p], semL[p])

            def prep(k, p):
                sl = chunk_slice(k)
                pltpu.make_async_copy(idxr.at[sl], rowc[p], semL[p]).wait()
                pltpu.make_async_copy(idxr.at[col_slice(k)], colc[p],
                                      semL[p]).wait()
                pltpu.make_async_copy(valr.at[sl], valc[p], semL[p]).wait()

                @pl.loop(0, _CH, step=16)
                def _(j):
                    js = pl.ds(j, 16)
                    r = rowc[p][js]
                    cc = colc[p][js]
                    i1[p][js] = r * n + cc
                    i2[p][js] = cc * n + r
                    colIdx[p][js] = cc
                    if with_rows:
                        rowIdx[p][js] = r

                pltpu.async_copy(maskf.at[i1[p]], m1b[p], semG[p])
                pltpu.async_copy(maskf.at[i2[p]], m2b[p], semG[p])
                if with_rows:
                    pltpu.async_copy(x.at[colIdx[p]], rowsb[p], semG[p])

            def process(k, p, store_guard):
                pltpu.make_async_copy(maskf.at[i1[p]], m1b[p], semG[p]).wait()
                pltpu.make_async_copy(maskf.at[i2[p]], m2b[p], semG[p]).wait()

                def drain_store():
                    pltpu.make_async_copy(wb[p], wout.at[pl.ds(0, _CH)],
                                          semS[p]).wait()

                if store_guard is True:
                    drain_store()
                else:
                    pl.when(store_guard)(drain_store)

                @pl.loop(0, _CH, step=16)
                def _(j):
                    js = pl.ds(j, 16)
                    sg1 = 1.0 / (1.0 + jnp.exp(-m1b[p][js]))
                    sg2 = 1.0 / (1.0 + jnp.exp(-m2b[p][js]))
                    wb[p][js] = valc[p][js] * ((sg1 + sg2) * 0.5)

                pltpu.async_copy(wb[p], wout.at[chunk_slice(k)], semS[p])

                if with_rows:
                    pltpu.make_async_copy(x.at[colIdx[p]], rowsb[p],
                                          semG[p]).wait()

                    @pl.loop(0, _CH, step=16)
                    def _(j):
                        wv = wb[p][pl.ds(j, 16)]
                        for i in range(16):
                            ws = wv[i]
                            for jj in range(d // 16):
                                rs = pl.ds(jj * 16, 16)
                                rowsb[p][j + i, rs] = rowsb[p][j + i, rs] * ws

                    pltpu.async_copy(rowsb[p], u_acc.at[rowIdx[p]], semA[p],
                                     add=True)
                else:
                    nv = nid_b[...]

                    @pl.loop(0, _CH, step=16)
                    def _(j):
                        js = pl.ds(j, 16)
                        sel = rowc[p][js] == nv
                        selw[p][js] = jnp.where(sel, wb[p][js], 0.0)

                    pltpu.async_copy(selw[p], s_acc.at[colIdx[p]], semA[p],
                                     add=True)

            def drain(p):
                if with_rows:
                    pltpu.make_async_copy(rowsb[p], u_acc.at[rowIdx[p]],
                                          semA[p]).wait()
                else:
                    pltpu.make_async_copy(selw[p], s_acc.at[colIdx[p]],
                                          semA[p]).wait()

            load(0, 0)
            load(1, 1)
            prep(0, 0)

            @pl.loop(0, nch - 2, step=3)
            def _(k):
                for q in range(3):
                    kk = k + q
                    load(kk + 2, (q + 2) % 3)
                    prep(kk + 1, (q + 1) % 3)
                    process(kk, q, kk >= 3)
                    pr = (q + 2) % 3

                    @pl.when(kk > 0)
                    def _():
                        drain(pr)

            prep(nch - 1, (nch - 1) % 3)
            process(nch - 2, (nch - 2) % 3, True)
            drain((nch - 3) % 3)
            process(nch - 1, (nch - 1) % 3, True)
            drain((nch - 2) % 3)
            drain((nch - 1) % 3)
            for p in range(3):
                pltpu.make_async_copy(wb[p], wout.at[pl.ds(0, _CH)],
                                      semS[p]).wait()

        run_phase(hidx, hval, horw, True)
        run_phase(vidx, vval, verw, False)

        plsc.subcore_barrier()
        pltpu.sync_copy(u_acc.at[pl.ds(r0, stripe)],
                        upart.at[c].at[pl.ds(r0, stripe)])

        @pl.when(sid == 0)
        def _():
            pltpu.sync_copy(s_acc, spart.at[c])

    return sc_kernel


def _tc_tail(u_part, s_part, W1, W2):
    n = u_part.shape[1]
    cdim = W2.shape[1]

    def body(u_ref, s_ref, w1_ref, w2_ref, o_ref):
        u = u_ref[0] + u_ref[1]
        g = jnp.dot(u, w1_ref[...], preferred_element_type=jnp.float32)
        g = jnp.maximum(g, 0.0)
        s = (s_ref[0] + s_ref[1]).reshape(n, 1)
        t = jnp.sum(g * s, axis=0, keepdims=True)
        p = jnp.dot(t, w2_ref[...], preferred_element_type=jnp.float32)
        m = jnp.max(p)
        ex = jnp.exp(p - m)
        o_ref[...] = ex / jnp.sum(ex)

    out = pl.pallas_call(
        body,
        out_shape=jax.ShapeDtypeStruct((1, cdim), jnp.float32),
    )(u_part, s_part, W1, W2)
    return out.reshape(cdim)


def kernel(mask, x, W1, W2, hor_values, hor_indices, ver_values, ver_indices,
           node_idx):
    n = mask.shape[0]
    e = hor_values.shape[0]
    d = x.shape[1]

    maskf = mask.reshape(n * n)
    nid = jnp.full((16,), node_idx, dtype=jnp.int32)

    sc = _sc_edge_kernel(n, e, d)
    hor_w, ver_w, u_part, s_part = sc(
        maskf, x,
        hor_indices.reshape(2 * e), hor_values,
        ver_indices.reshape(2 * e), ver_values,
        nid)

    res = _tc_tail(u_part, s_part, W1, W2)
    return (res, hor_w, ver_w)
